# baseline (speedup 1.0000x reference)
import functools
import math

import jax
import jax.numpy as jnp
from jax import lax
from jax.experimental import pallas as pl
from jax.experimental.pallas import tpu as pltpu
from jax.experimental.pallas import tpu_sc as plsc

N = 10000
NFEAT = 128
NLABEL = 40
NHID = 256
CNT_COL = 40

NPAD = 10240
ROWS_PER_TILE = NPAD // 16
CHUNK = 128
ZCH = 128
NC2 = 160
EPT = NC2 * CHUNK
E2 = 16 * EPT
NBUF = 2

RB = 1000
NB = N // RB

_F32 = jnp.float32


GRP = 16
NGRP = NC2 // GRP


def _make_sc_pass():
    mesh = plsc.VectorSubcoreMesh(core_axis_name="c", subcore_axis_name="s",
                                  num_cores=2, num_subcores=16)

    def body(tbl, srcc, dstc, zrows, out, acc,
             b0, b1, si0, si1, di0, di1,
             gs0, gs1, ss0, ss1, is0, is1, id0, id1):
        bufs = (b0, b1)
        si = (si0, si1)
        di = (di0, di1)
        gsem = (gs0, gs1)
        ssem = (ss0, ss1)
        isem = (is0, is1)
        idsem = (id0, id1)
        c = lax.axis_index("c")
        s = lax.axis_index("s")
        slab = s * ROWS_PER_TILE

        for k in range(ROWS_PER_TILE // ZCH):
            pltpu.sync_copy(zrows, acc.at[pl.ds(slab + k * ZCH, ZCH)])
        plsc.subcore_barrier()

        def load_idx(g, p):
            pltpu.async_copy(srcc.at[c, s, g], si[p], isem[p])
            pltpu.async_copy(dstc.at[s, g], di[p], idsem[p])

        def wait_idx(p):
            pltpu.make_async_copy(srcc.at[0, 0, 0], si[p], isem[p]).wait()
            pltpu.make_async_copy(dstc.at[0, 0], di[p], idsem[p]).wait()

        def wait_g(b):
            pltpu.make_async_copy(tbl.at[pl.ds(0, CHUNK)], bufs[b],
                                  gsem[b]).wait()

        def wait_s(b):
            pltpu.make_async_copy(tbl.at[pl.ds(0, CHUNK)], bufs[b],
                                  ssem[b]).wait()

        def do_group(p, g, first=False):
            wait_idx(p)
            load_idx(jnp.minimum(g + 1, NGRP - 1), 1 - p)
            for b8 in range(GRP):
                b = b8 % NBUF
                if not (first and b8 < NBUF):
                    wait_s(b)
                pltpu.async_copy(tbl.at[si[p].at[b8]], bufs[b], gsem[b])
                wait_g(b)
                pltpu.async_copy(bufs[b], acc.at[di[p].at[b8]], ssem[b],
                                 add=True)

        load_idx(0, 0)
        do_group(0, 0, first=True)
        do_group(1, 1)

        def pair(j, carry):
            do_group(0, 2 * j)
            do_group(1, 2 * j + 1)
            return carry

        lax.fori_loop(1, NGRP // 2, pair, 0)
        for b in range(NBUF):
            wait_s(b)
        wait_idx(0)

        plsc.subcore_barrier()
        pltpu.sync_copy(acc.at[pl.ds(slab, ROWS_PER_TILE)],
                        out.at[c, pl.ds(slab, ROWS_PER_TILE)])

    return pl.kernel(
        body,
        out_type=jax.ShapeDtypeStruct((2, NPAD, 128), _F32),
        mesh=mesh,
        scratch_types=[
            pltpu.VMEM_SHARED((NPAD, 128), _F32),
            pltpu.VMEM((CHUNK, 128), _F32),
            pltpu.VMEM((CHUNK, 128), _F32),
            pltpu.VMEM((GRP, CHUNK), jnp.int32),
            pltpu.VMEM((GRP, CHUNK), jnp.int32),
            pltpu.VMEM((GRP, CHUNK), jnp.int32),
            pltpu.VMEM((GRP, CHUNK), jnp.int32),
        ] + [pltpu.SemaphoreType.DMA] * 8,
    )


@functools.lru_cache(maxsize=None)
def _sc_pass_cached():
    return _make_sc_pass()


def _sc_pass2(tbl, srcc, dstc, zrows):
    return _sc_pass_cached()(tbl, srcc, dstc, zrows)


def _te_body(xs_ref, wt1_ref, bt1_ref, wt2_ref, bt2_ref, out_ref):
    col = lax.broadcasted_iota(jnp.int32, (1, 128), 1)
    k = jnp.where(col < 64, col, col - 64).astype(_F32)
    freq = jnp.exp(k * (-(math.log(10000.0) / 63.0)))
    z = xs_ref[0, 0] * freq
    e = jnp.where(col < 64, jnp.sin(z), jnp.cos(z))
    u = jnp.dot(e, wt1_ref[...], preferred_element_type=_F32) + bt1_ref[...]
    u = jnp.where(u > 0, u, jnp.exp(jnp.minimum(u, 0.0)) - 1.0)
    te = jnp.dot(u, wt2_ref[...], preferred_element_type=_F32) + bt2_ref[...]
    out_ref[...] = jnp.broadcast_to(te, (8, NHID))


def _time_embedding(xs, Wt1, bt1, Wt2, bt2):
    return pl.pallas_call(
        _te_body,
        out_shape=jax.ShapeDtypeStruct((8, NHID), _F32),
    )(xs, Wt1, bt1.reshape(1, 128), Wt2, bt2.reshape(1, NHID))


def _pre1_body(x_ref, q_ref, wra_ref, wrq_ref, te_ref, out_ref):
    acc = jnp.dot(x_ref[...], wra_ref[...], preferred_element_type=_F32)
    acc += jnp.dot(q_ref[...], wrq_ref[...], preferred_element_type=_F32)
    out_ref[0] = acc + te_ref[0:1, :]


def _pre1(X, Q, Wra, Wrq, te):
    return pl.pallas_call(
        _pre1_body,
        grid=(NB, 2),
        in_specs=[
            pl.BlockSpec((RB, 128), lambda i, j: (i, 0)),
            pl.BlockSpec((RB, 128), lambda i, j: (i, 0)),
            pl.BlockSpec((128, 128), lambda i, j: (0, j)),
            pl.BlockSpec((128, 128), lambda i, j: (0, j)),
            pl.BlockSpec((8, 128), lambda i, j: (0, j)),
        ],
        out_specs=pl.BlockSpec((1, RB, 128), lambda i, j: (j, i, 0)),
        out_shape=jax.ShapeDtypeStruct((2, N, 128), _F32),
    )(X, Q, Wra, Wrq, te)


def _pre23_body(r0_ref, r1_ref, q_ref, aq_ref,
                wrr0_ref, wrr1_ref, wrq_ref, wlq_ref, te_ref, out_ref):
    aq = aq_ref[0]
    inv = 1.0 / jnp.maximum(aq[:, CNT_COL:CNT_COL + 1], 1.0)
    acc = jnp.dot(r0_ref[0], wrr0_ref[...], preferred_element_type=_F32)
    acc += jnp.dot(r1_ref[0], wrr1_ref[...], preferred_element_type=_F32)
    acc += jnp.dot(q_ref[...], wrq_ref[...], preferred_element_type=_F32)
    acc += jnp.dot(aq * inv, wlq_ref[...], preferred_element_type=_F32)
    out_ref[0] = acc + te_ref[0:1, :]


def _pre23(R, Q, accP0, Wrr, Wrq, Wlq, te):
    return pl.pallas_call(
        _pre23_body,
        grid=(NB, 2),
        in_specs=[
            pl.BlockSpec((1, RB, 128), lambda i, j: (0, i, 0)),
            pl.BlockSpec((1, RB, 128), lambda i, j: (1, i, 0)),
            pl.BlockSpec((RB, 128), lambda i, j: (i, 0)),
            pl.BlockSpec((1, RB, 128), lambda i, j: (1, i, 0)),
            pl.BlockSpec((128, 128), lambda i, j: (0, j)),
            pl.BlockSpec((128, 128), lambda i, j: (1, j)),
            pl.BlockSpec((128, 128), lambda i, j: (0, j)),
            pl.BlockSpec((128, 128), lambda i, j: (0, j)),
            pl.BlockSpec((8, 128), lambda i, j: (0, j)),
        ],
        out_specs=pl.BlockSpec((1, RB, 128), lambda i, j: (j, i, 0)),
        out_shape=jax.ShapeDtypeStruct((2, N, 128), _F32),
    )(R, R, Q, accP0, Wrr, Wrr, Wrq, Wlq, te)


def _combine1_body(ax_ref, aq_ref, pre_ref, wla_ref, wlq_ref, out_ref):
    aq = aq_ref[0]
    inv = 1.0 / jnp.maximum(aq[:, CNT_COL:CNT_COL + 1], 1.0)
    acc = jnp.dot(ax_ref[0] * inv, wla_ref[...], preferred_element_type=_F32)
    acc += jnp.dot(aq * inv, wlq_ref[...], preferred_element_type=_F32)
    out_ref[0] = jnp.maximum(acc + pre_ref[0], 0.0)


def _combine1(accP0, pre, Wla, Wlq):
    return pl.pallas_call(
        _combine1_body,
        grid=(NB, 2),
        in_specs=[
            pl.BlockSpec((1, RB, 128), lambda i, j: (0, i, 0)),
            pl.BlockSpec((1, RB, 128), lambda i, j: (1, i, 0)),
            pl.BlockSpec((1, RB, 128), lambda i, j: (j, i, 0)),
            pl.BlockSpec((128, 128), lambda i, j: (0, j)),
            pl.BlockSpec((128, 128), lambda i, j: (0, j)),
        ],
        out_specs=pl.BlockSpec((1, RB, 128), lambda i, j: (j, i, 0)),
        out_shape=jax.ShapeDtypeStruct((2, N, 128), _F32),
    )(accP0, accP0, pre, Wla, Wlq)


def _combine2_body(p0_ref, p1_ref, aq_ref, pre_ref,
                   wlr0_ref, wlr1_ref, out_ref):
    inv = 1.0 / jnp.maximum(aq_ref[0][:, CNT_COL:CNT_COL + 1], 1.0)
    acc = jnp.dot(p0_ref[0] * inv, wlr0_ref[...], preferred_element_type=_F32)
    acc += jnp.dot(p1_ref[0] * inv, wlr1_ref[...],
                   preferred_element_type=_F32)
    out_ref[0] = jnp.maximum(acc + pre_ref[0], 0.0)


def _combine2(accP, accP0, pre, Wlr):
    return pl.pallas_call(
        _combine2_body,
        grid=(NB, 2),
        in_specs=[
            pl.BlockSpec((1, RB, 128), lambda i, j: (0, i, 0)),
            pl.BlockSpec((1, RB, 128), lambda i, j: (1, i, 0)),
            pl.BlockSpec((1, RB, 128), lambda i, j: (1, i, 0)),
            pl.BlockSpec((1, RB, 128), lambda i, j: (j, i, 0)),
            pl.BlockSpec((128, 128), lambda i, j: (0, j)),
            pl.BlockSpec((128, 128), lambda i, j: (1, j)),
        ],
        out_specs=pl.BlockSpec((1, RB, 128), lambda i, j: (j, i, 0)),
        out_shape=jax.ShapeDtypeStruct((2, N, 128), _F32),
    )(accP, accP, accP0, pre, Wlr, Wlr)


def _c3f_body(p0_ref, p1_ref, aq_ref, pre0_ref, pre1_ref, q_ref, wlr_ref,
              wf1r_ref, wf1q_ref, bf1_ref, wf2_ref, bf2_ref, out_ref):
    inv = 1.0 / jnp.maximum(aq_ref[0][:, CNT_COL:CNT_COL + 1], 1.0)
    acc = jnp.dot(p0_ref[0] * inv, wlr_ref[0:128],
                  preferred_element_type=_F32)
    acc += jnp.dot(p1_ref[0] * inv, wlr_ref[128:256],
                   preferred_element_type=_F32)
    acc += jnp.concatenate([pre0_ref[0], pre1_ref[0]], axis=1)
    r3 = jnp.maximum(acc, 0.0)
    u = jnp.dot(r3, wf1r_ref[...], preferred_element_type=_F32)
    u += jnp.dot(q_ref[...], wf1q_ref[...], preferred_element_type=_F32)
    u += bf1_ref[...]
    u = jnp.where(u > 0, u, jnp.exp(jnp.minimum(u, 0.0)) - 1.0)
    out_ref[...] = jnp.dot(u, wf2_ref[...], preferred_element_type=_F32) \
        + bf2_ref[...]


def _combine3_final(accP, accP0, pre, Q, Wlr, Wf1r, Wf1q, bf1, Wf2, bf2):
    fdim2 = 2 * (NHID + NLABEL)
    return pl.pallas_call(
        _c3f_body,
        grid=(NB,),
        in_specs=[
            pl.BlockSpec((1, RB, 128), lambda i: (0, i, 0)),
            pl.BlockSpec((1, RB, 128), lambda i: (1, i, 0)),
            pl.BlockSpec((1, RB, 128), lambda i: (1, i, 0)),
            pl.BlockSpec((1, RB, 128), lambda i: (0, i, 0)),
            pl.BlockSpec((1, RB, 128), lambda i: (1, i, 0)),
            pl.BlockSpec((RB, 128), lambda i: (i, 0)),
            pl.BlockSpec((NHID, NHID), lambda i: (0, 0)),
            pl.BlockSpec((NHID, fdim2), lambda i: (0, 0)),
            pl.BlockSpec((128, fdim2), lambda i: (0, 0)),
            pl.BlockSpec((1, fdim2), lambda i: (0, 0)),
            pl.BlockSpec((fdim2, NLABEL), lambda i: (0, 0)),
            pl.BlockSpec((1, NLABEL), lambda i: (0, 0)),
        ],
        out_specs=pl.BlockSpec((RB, NLABEL), lambda i: (i, 0)),
        out_shape=jax.ShapeDtypeStruct((N, NLABEL), _F32),
    )(accP, accP, accP0, pre, pre, Q, Wlr,
      Wf1r, Wf1q, bf1, Wf2, bf2)


def _pad_q_rows(Wq):
    return jnp.concatenate(
        [Wq, jnp.zeros((128 - NLABEL, Wq.shape[1]), _F32)], axis=0)


def kernel(x, q_Y_sample, edge_index, t, num_steps,
           Wl0, bl0, Wr0, Wl1, bl1, Wr1, Wl2, bl2, Wr2,
           Wt1, bt1, Wt2, bt2, Wf1, bf1, Wf2, bf2):
    npad_e = E2 - edge_index.shape[1]
    iota = jnp.arange(npad_e, dtype=jnp.int32)
    srcp = jnp.concatenate([edge_index[0], (iota * 37) % N])
    dstp = jnp.concatenate([edge_index[1], N + (iota % (NPAD - N))])
    srcc = jnp.stack([srcp, srcp + N]).reshape(2, 16, NGRP, GRP, CHUNK)
    dstc = dstp.reshape(16, NGRP, GRP, CHUNK)

    Q = jnp.concatenate(
        [q_Y_sample, jnp.ones((N, 1), _F32),
         jnp.zeros((N, 128 - NLABEL - 1), _F32)], axis=1)
    XQ = jnp.concatenate([x, Q], axis=0)

    zrows = jnp.zeros((ZCH, 128), _F32)

    Wl0a, Wl0q = Wl0[:NFEAT], _pad_q_rows(Wl0[NFEAT:])
    Wr0a, Wr0q = Wr0[:NFEAT], _pad_q_rows(Wr0[NFEAT:])
    Wl1r, Wl1q = Wl1[:NHID], _pad_q_rows(Wl1[NHID:])
    Wr1r, Wr1q = Wr1[:NHID], _pad_q_rows(Wr1[NHID:])
    Wl2r, Wl2q = Wl2[:NHID], _pad_q_rows(Wl2[NHID:])
    Wr2r, Wr2q = Wr2[:NHID], _pad_q_rows(Wr2[NHID:])
    Wf1r, Wf1q = Wf1[:NHID], _pad_q_rows(Wf1[NHID:])

    ns = jnp.asarray(num_steps, _F32)
    xs = (t / ns * ns * 4.0).reshape(1, 1)
    te = _time_embedding(xs, Wt1, bt1, Wt2, bt2)
    te0 = te + bl0.reshape(1, NHID)
    te1 = te + bl1.reshape(1, NHID)
    te2 = te + bl2.reshape(1, NHID)

    accP0 = _sc_pass2(XQ, srcc, dstc, zrows)
    pre1 = _pre1(x, Q, Wr0a, Wr0q, te0)
    r1 = _combine1(accP0, pre1, Wl0a, Wl0q)
    acc1 = _sc_pass2(r1.reshape(2 * N, 128), srcc, dstc, zrows)
    pre2 = _pre23(r1, Q, accP0, Wr1r, Wr1q, Wl1q, te1)
    r2 = _combine2(acc1, accP0, pre2, Wl1r)
    acc2 = _sc_pass2(r2.reshape(2 * N, 128), srcc, dstc, zrows)
    pre3 = _pre23(r2, Q, accP0, Wr2r, Wr2q, Wl2q, te2)
    return _combine3_final(acc2, accP0, pre3, Q, Wl2r,
                           Wf1r, Wf1q, bf1.reshape(1, -1),
                           Wf2, bf2.reshape(1, -1))


# speedup vs baseline: 1.0032x; 1.0032x over previous
import functools
import math

import jax
import jax.numpy as jnp
from jax import lax
from jax.experimental import pallas as pl
from jax.experimental.pallas import tpu as pltpu
from jax.experimental.pallas import tpu_sc as plsc

N = 10000
NFEAT = 128
NLABEL = 40
NHID = 256
CNT_COL = 40

NPAD = 10240
ROWS_PER_TILE = NPAD // 16
CHUNK = 128
ZCH = 128
NC2 = 160
EPT = NC2 * CHUNK
E2 = 16 * EPT
NBUF = 2

RB = 1000
NB = N // RB

_F32 = jnp.float32


GRP = 8
NGRP = NC2 // GRP


def _make_sc_pass():
    mesh = plsc.VectorSubcoreMesh(core_axis_name="c", subcore_axis_name="s",
                                  num_cores=2, num_subcores=16)

    def body(tbl, srcc, dstc, zrows, out, acc,
             b0, b1, si0, si1, di0, di1,
             gs0, gs1, ss0, ss1, is0, is1, id0, id1):
        bufs = (b0, b1)
        si = (si0, si1)
        di = (di0, di1)
        gsem = (gs0, gs1)
        ssem = (ss0, ss1)
        isem = (is0, is1)
        idsem = (id0, id1)
        c = lax.axis_index("c")
        s = lax.axis_index("s")
        slab = s * ROWS_PER_TILE

        for k in range(ROWS_PER_TILE // ZCH):
            pltpu.sync_copy(zrows, acc.at[pl.ds(slab + k * ZCH, ZCH)])
        plsc.subcore_barrier()

        def load_idx(g, p):
            pltpu.async_copy(srcc.at[c, s, g], si[p], isem[p])
            pltpu.async_copy(dstc.at[s, g], di[p], idsem[p])

        def wait_idx(p):
            pltpu.make_async_copy(srcc.at[0, 0, 0], si[p], isem[p]).wait()
            pltpu.make_async_copy(dstc.at[0, 0], di[p], idsem[p]).wait()

        def wait_g(b):
            pltpu.make_async_copy(tbl.at[pl.ds(0, CHUNK)], bufs[b],
                                  gsem[b]).wait()

        def wait_s(b):
            pltpu.make_async_copy(tbl.at[pl.ds(0, CHUNK)], bufs[b],
                                  ssem[b]).wait()

        def do_group(p, g, first=False):
            wait_idx(p)
            load_idx(jnp.minimum(g + 1, NGRP - 1), 1 - p)
            for b8 in range(GRP):
                b = b8 % NBUF
                if not (first and b8 < NBUF):
                    wait_s(b)
                pltpu.async_copy(tbl.at[si[p].at[b8]], bufs[b], gsem[b])
                wait_g(b)
                pltpu.async_copy(bufs[b], acc.at[di[p].at[b8]], ssem[b],
                                 add=True)

        load_idx(0, 0)
        do_group(0, 0, first=True)
        do_group(1, 1)

        def pair(j, carry):
            do_group(0, 2 * j)
            do_group(1, 2 * j + 1)
            return carry

        lax.fori_loop(1, NGRP // 2, pair, 0)
        for b in range(NBUF):
            wait_s(b)
        wait_idx(0)

        plsc.subcore_barrier()
        pltpu.sync_copy(acc.at[pl.ds(slab, ROWS_PER_TILE)],
                        out.at[c, pl.ds(slab, ROWS_PER_TILE)])

    return pl.kernel(
        body,
        out_type=jax.ShapeDtypeStruct((2, NPAD, 128), _F32),
        mesh=mesh,
        scratch_types=[
            pltpu.VMEM_SHARED((NPAD, 128), _F32),
            pltpu.VMEM((CHUNK, 128), _F32),
            pltpu.VMEM((CHUNK, 128), _F32),
            pltpu.VMEM((GRP, CHUNK), jnp.int32),
            pltpu.VMEM((GRP, CHUNK), jnp.int32),
            pltpu.VMEM((GRP, CHUNK), jnp.int32),
            pltpu.VMEM((GRP, CHUNK), jnp.int32),
        ] + [pltpu.SemaphoreType.DMA] * 8,
    )


@functools.lru_cache(maxsize=None)
def _sc_pass_cached():
    return _make_sc_pass()


def _sc_pass2(tbl, srcc, dstc, zrows):
    return _sc_pass_cached()(tbl, srcc, dstc, zrows)


def _te_body(xs_ref, wt1_ref, bt1_ref, wt2_ref, bt2_ref, out_ref):
    col = lax.broadcasted_iota(jnp.int32, (1, 128), 1)
    k = jnp.where(col < 64, col, col - 64).astype(_F32)
    freq = jnp.exp(k * (-(math.log(10000.0) / 63.0)))
    z = xs_ref[0, 0] * freq
    e = jnp.where(col < 64, jnp.sin(z), jnp.cos(z))
    u = jnp.dot(e, wt1_ref[...], preferred_element_type=_F32) + bt1_ref[...]
    u = jnp.where(u > 0, u, jnp.exp(jnp.minimum(u, 0.0)) - 1.0)
    te = jnp.dot(u, wt2_ref[...], preferred_element_type=_F32) + bt2_ref[...]
    out_ref[...] = jnp.broadcast_to(te, (8, NHID))


def _time_embedding(xs, Wt1, bt1, Wt2, bt2):
    return pl.pallas_call(
        _te_body,
        out_shape=jax.ShapeDtypeStruct((8, NHID), _F32),
    )(xs, Wt1, bt1.reshape(1, 128), Wt2, bt2.reshape(1, NHID))


def _pre1_body(x_ref, q_ref, wra_ref, wrq_ref, te_ref, out_ref):
    acc = jnp.dot(x_ref[...], wra_ref[...], preferred_element_type=_F32)
    acc += jnp.dot(q_ref[...], wrq_ref[...], preferred_element_type=_F32)
    out_ref[0] = acc + te_ref[0:1, :]


def _pre1(X, Q, Wra, Wrq, te):
    return pl.pallas_call(
        _pre1_body,
        grid=(NB, 2),
        in_specs=[
            pl.BlockSpec((RB, 128), lambda i, j: (i, 0)),
            pl.BlockSpec((RB, 128), lambda i, j: (i, 0)),
            pl.BlockSpec((128, 128), lambda i, j: (0, j)),
            pl.BlockSpec((128, 128), lambda i, j: (0, j)),
            pl.BlockSpec((8, 128), lambda i, j: (0, j)),
        ],
        out_specs=pl.BlockSpec((1, RB, 128), lambda i, j: (j, i, 0)),
        out_shape=jax.ShapeDtypeStruct((2, N, 128), _F32),
    )(X, Q, Wra, Wrq, te)


def _pre23_body(r0_ref, r1_ref, q_ref, aq_ref,
                wrr0_ref, wrr1_ref, wrq_ref, wlq_ref, te_ref, out_ref):
    aq = aq_ref[0]
    inv = 1.0 / jnp.maximum(aq[:, CNT_COL:CNT_COL + 1], 1.0)
    acc = jnp.dot(r0_ref[0], wrr0_ref[...], preferred_element_type=_F32)
    acc += jnp.dot(r1_ref[0], wrr1_ref[...], preferred_element_type=_F32)
    acc += jnp.dot(q_ref[...], wrq_ref[...], preferred_element_type=_F32)
    acc += jnp.dot(aq * inv, wlq_ref[...], preferred_element_type=_F32)
    out_ref[0] = acc + te_ref[0:1, :]


def _pre23(R, Q, accP0, Wrr, Wrq, Wlq, te):
    return pl.pallas_call(
        _pre23_body,
        grid=(NB, 2),
        in_specs=[
            pl.BlockSpec((1, RB, 128), lambda i, j: (0, i, 0)),
            pl.BlockSpec((1, RB, 128), lambda i, j: (1, i, 0)),
            pl.BlockSpec((RB, 128), lambda i, j: (i, 0)),
            pl.BlockSpec((1, RB, 128), lambda i, j: (1, i, 0)),
            pl.BlockSpec((128, 128), lambda i, j: (0, j)),
            pl.BlockSpec((128, 128), lambda i, j: (1, j)),
            pl.BlockSpec((128, 128), lambda i, j: (0, j)),
            pl.BlockSpec((128, 128), lambda i, j: (0, j)),
            pl.BlockSpec((8, 128), lambda i, j: (0, j)),
        ],
        out_specs=pl.BlockSpec((1, RB, 128), lambda i, j: (j, i, 0)),
        out_shape=jax.ShapeDtypeStruct((2, N, 128), _F32),
    )(R, R, Q, accP0, Wrr, Wrr, Wrq, Wlq, te)


def _combine1_body(ax_ref, aq_ref, pre_ref, wla_ref, wlq_ref, out_ref):
    aq = aq_ref[0]
    inv = 1.0 / jnp.maximum(aq[:, CNT_COL:CNT_COL + 1], 1.0)
    acc = jnp.dot(ax_ref[0] * inv, wla_ref[...], preferred_element_type=_F32)
    acc += jnp.dot(aq * inv, wlq_ref[...], preferred_element_type=_F32)
    out_ref[0] = jnp.maximum(acc + pre_ref[0], 0.0)


def _combine1(accP0, pre, Wla, Wlq):
    return pl.pallas_call(
        _combine1_body,
        grid=(NB, 2),
        in_specs=[
            pl.BlockSpec((1, RB, 128), lambda i, j: (0, i, 0)),
            pl.BlockSpec((1, RB, 128), lambda i, j: (1, i, 0)),
            pl.BlockSpec((1, RB, 128), lambda i, j: (j, i, 0)),
            pl.BlockSpec((128, 128), lambda i, j: (0, j)),
            pl.BlockSpec((128, 128), lambda i, j: (0, j)),
        ],
        out_specs=pl.BlockSpec((1, RB, 128), lambda i, j: (j, i, 0)),
        out_shape=jax.ShapeDtypeStruct((2, N, 128), _F32),
    )(accP0, accP0, pre, Wla, Wlq)


def _combine2_body(p0_ref, p1_ref, aq_ref, pre_ref,
                   wlr0_ref, wlr1_ref, out_ref):
    inv = 1.0 / jnp.maximum(aq_ref[0][:, CNT_COL:CNT_COL + 1], 1.0)
    acc = jnp.dot(p0_ref[0] * inv, wlr0_ref[...], preferred_element_type=_F32)
    acc += jnp.dot(p1_ref[0] * inv, wlr1_ref[...],
                   preferred_element_type=_F32)
    out_ref[0] = jnp.maximum(acc + pre_ref[0], 0.0)


def _combine2(accP, accP0, pre, Wlr):
    return pl.pallas_call(
        _combine2_body,
        grid=(NB, 2),
        in_specs=[
            pl.BlockSpec((1, RB, 128), lambda i, j: (0, i, 0)),
            pl.BlockSpec((1, RB, 128), lambda i, j: (1, i, 0)),
            pl.BlockSpec((1, RB, 128), lambda i, j: (1, i, 0)),
            pl.BlockSpec((1, RB, 128), lambda i, j: (j, i, 0)),
            pl.BlockSpec((128, 128), lambda i, j: (0, j)),
            pl.BlockSpec((128, 128), lambda i, j: (1, j)),
        ],
        out_specs=pl.BlockSpec((1, RB, 128), lambda i, j: (j, i, 0)),
        out_shape=jax.ShapeDtypeStruct((2, N, 128), _F32),
    )(accP, accP, accP0, pre, Wlr, Wlr)


def _c3f_body(p0_ref, p1_ref, aq_ref, pre0_ref, pre1_ref, q_ref, wlr_ref,
              wf1r_ref, wf1q_ref, bf1_ref, wf2_ref, bf2_ref, out_ref):
    inv = 1.0 / jnp.maximum(aq_ref[0][:, CNT_COL:CNT_COL + 1], 1.0)
    acc = jnp.dot(p0_ref[0] * inv, wlr_ref[0:128],
                  preferred_element_type=_F32)
    acc += jnp.dot(p1_ref[0] * inv, wlr_ref[128:256],
                   preferred_element_type=_F32)
    acc += jnp.concatenate([pre0_ref[0], pre1_ref[0]], axis=1)
    r3 = jnp.maximum(acc, 0.0)
    u = jnp.dot(r3, wf1r_ref[...], preferred_element_type=_F32)
    u += jnp.dot(q_ref[...], wf1q_ref[...], preferred_element_type=_F32)
    u += bf1_ref[...]
    u = jnp.where(u > 0, u, jnp.exp(jnp.minimum(u, 0.0)) - 1.0)
    out_ref[...] = jnp.dot(u, wf2_ref[...], preferred_element_type=_F32) \
        + bf2_ref[...]


def _combine3_final(accP, accP0, pre, Q, Wlr, Wf1r, Wf1q, bf1, Wf2, bf2):
    fdim2 = 2 * (NHID + NLABEL)
    return pl.pallas_call(
        _c3f_body,
        grid=(NB,),
        in_specs=[
            pl.BlockSpec((1, RB, 128), lambda i: (0, i, 0)),
            pl.BlockSpec((1, RB, 128), lambda i: (1, i, 0)),
            pl.BlockSpec((1, RB, 128), lambda i: (1, i, 0)),
            pl.BlockSpec((1, RB, 128), lambda i: (0, i, 0)),
            pl.BlockSpec((1, RB, 128), lambda i: (1, i, 0)),
            pl.BlockSpec((RB, 128), lambda i: (i, 0)),
            pl.BlockSpec((NHID, NHID), lambda i: (0, 0)),
            pl.BlockSpec((NHID, fdim2), lambda i: (0, 0)),
            pl.BlockSpec((128, fdim2), lambda i: (0, 0)),
            pl.BlockSpec((1, fdim2), lambda i: (0, 0)),
            pl.BlockSpec((fdim2, NLABEL), lambda i: (0, 0)),
            pl.BlockSpec((1, NLABEL), lambda i: (0, 0)),
        ],
        out_specs=pl.BlockSpec((RB, NLABEL), lambda i: (i, 0)),
        out_shape=jax.ShapeDtypeStruct((N, NLABEL), _F32),
    )(accP, accP, accP0, pre, pre, Q, Wlr,
      Wf1r, Wf1q, bf1, Wf2, bf2)


def _pad_q_rows(Wq):
    return jnp.concatenate(
        [Wq, jnp.zeros((128 - NLABEL, Wq.shape[1]), _F32)], axis=0)


def kernel(x, q_Y_sample, edge_index, t, num_steps,
           Wl0, bl0, Wr0, Wl1, bl1, Wr1, Wl2, bl2, Wr2,
           Wt1, bt1, Wt2, bt2, Wf1, bf1, Wf2, bf2):
    npad_e = E2 - edge_index.shape[1]
    iota = jnp.arange(npad_e, dtype=jnp.int32)
    srcp = jnp.concatenate([edge_index[0], (iota * 37) % N])
    dstp = jnp.concatenate([edge_index[1], N + (iota % (NPAD - N))])
    srcc = jnp.stack([srcp, srcp + N]).reshape(2, 16, NGRP, GRP, CHUNK)
    dstc = dstp.reshape(16, NGRP, GRP, CHUNK)

    Q = jnp.concatenate(
        [q_Y_sample, jnp.ones((N, 1), _F32),
         jnp.zeros((N, 128 - NLABEL - 1), _F32)], axis=1)
    XQ = jnp.concatenate([x, Q], axis=0)

    zrows = jnp.zeros((ZCH, 128), _F32)

    Wl0a, Wl0q = Wl0[:NFEAT], _pad_q_rows(Wl0[NFEAT:])
    Wr0a, Wr0q = Wr0[:NFEAT], _pad_q_rows(Wr0[NFEAT:])
    Wl1r, Wl1q = Wl1[:NHID], _pad_q_rows(Wl1[NHID:])
    Wr1r, Wr1q = Wr1[:NHID], _pad_q_rows(Wr1[NHID:])
    Wl2r, Wl2q = Wl2[:NHID], _pad_q_rows(Wl2[NHID:])
    Wr2r, Wr2q = Wr2[:NHID], _pad_q_rows(Wr2[NHID:])
    Wf1r, Wf1q = Wf1[:NHID], _pad_q_rows(Wf1[NHID:])

    ns = jnp.asarray(num_steps, _F32)
    xs = (t / ns * ns * 4.0).reshape(1, 1)
    te = _time_embedding(xs, Wt1, bt1, Wt2, bt2)
    te0 = te + bl0.reshape(1, NHID)
    te1 = te + bl1.reshape(1, NHID)
    te2 = te + bl2.reshape(1, NHID)

    accP0 = _sc_pass2(XQ, srcc, dstc, zrows)
    pre1 = _pre1(x, Q, Wr0a, Wr0q, te0)
    r1 = _combine1(accP0, pre1, Wl0a, Wl0q)
    acc1 = _sc_pass2(r1.reshape(2 * N, 128), srcc, dstc, zrows)
    pre2 = _pre23(r1, Q, accP0, Wr1r, Wr1q, Wl1q, te1)
    r2 = _combine2(acc1, accP0, pre2, Wl1r)
    acc2 = _sc_pass2(r2.reshape(2 * N, 128), srcc, dstc, zrows)
    pre3 = _pre23(r2, Q, accP0, Wr2r, Wr2q, Wl2q, te2)
    return _combine3_final(acc2, accP0, pre3, Q, Wl2r,
                           Wf1r, Wf1q, bf1.reshape(1, -1),
                           Wf2, bf2.reshape(1, -1))
